# baseline (speedup 1.0000x reference)
import jax
import jax.numpy as jnp
from jax.experimental import pallas as pl
from jax.experimental.pallas import tpu as pltpu
from jax.experimental.pallas import tpu_sc as plsc

_K = 8192
_D = 256
_TT = 512
_N_TOK = 4608
_NT = _N_TOK // _TT
_HW = 576
_B = 8
_COMMIT = 0.25


def _bnorm_body(e_ref, b_ref):
    e = e_ref[...]
    b_ref[...] = jnp.sum(e * e, axis=1)[None, :]


def _bnorm(embed):
    return pl.pallas_call(
        _bnorm_body,
        grid=(4,),
        in_specs=[pl.BlockSpec((_K // 4, _D), lambda i: (i, 0))],
        out_specs=pl.BlockSpec((1, _K // 4), lambda i: (0, i)),
        out_shape=jax.ShapeDtypeStruct((1, _K), jnp.float32),
    )(embed)


def _dist_body(x_ref, e_ref, b_ref, idx_ref, loss_ref, m_ref):
    x = x_ref[...]
    a = jnp.sum(x * x, axis=1, keepdims=True)
    x2 = x + x
    kt_sz = 512
    for kt in range(_K // kt_sz):
        sk = slice(kt * kt_sz, (kt + 1) * kt_sz)
        m_ref[:, sk] = jax.lax.dot_general(
            x2, e_ref[sk, :], (((1,), (1,)), ((), ())),
            preferred_element_type=jnp.float32)

    nslice = _K // 128
    colmin = (a + b_ref[:, 0:128]) - m_ref[:, 0:128]
    colidx = jnp.zeros((_TT, 128), jnp.int32)
    for j in range(1, nslice):
        sl = slice(j * 128, (j + 1) * 128)
        dj = (a + b_ref[:, sl]) - m_ref[:, sl]
        lt = dj < colmin
        colmin = jnp.minimum(colmin, dj)
        colidx = jnp.where(lt, j, colidx)
    lane = jax.lax.broadcasted_iota(jnp.int32, (_TT, 128), 1)
    g = colidx * 128 + lane
    dmin = jnp.min(colmin, axis=1, keepdims=True)
    idx = jnp.min(jnp.where(colmin == dmin, g, _K), axis=1)
    idx_ref[0, 0, :] = idx
    loss_ref[...] = jnp.sum(dmin, axis=0, keepdims=True)[None]


def _compute_indices(flat, embed, b2d):
    nt = flat.shape[0] // _TT
    idx3, loss = pl.pallas_call(
        _dist_body,
        grid=(nt,),
        in_specs=[
            pl.BlockSpec((_TT, _D), lambda i: (i, 0)),
            pl.BlockSpec((_K, _D), lambda i: (0, 0)),
            pl.BlockSpec((1, _K), lambda i: (0, 0)),
        ],
        out_specs=[
            pl.BlockSpec((1, 1, _TT), lambda i: (i, 0, 0)),
            pl.BlockSpec((1, 1, 1), lambda i: (i, 0, 0)),
        ],
        out_shape=[
            jax.ShapeDtypeStruct((nt, 1, _TT), jnp.int32),
            jax.ShapeDtypeStruct((nt, 1, 1), jnp.float32),
        ],
        scratch_shapes=[pltpu.VMEM((_TT, _K), jnp.float32)],
    )(flat, embed, b2d)
    return idx3.reshape(nt * _TT), loss.reshape(1, nt)


_GW = 128


def _sc_gather(embed, idx):
    n = idx.shape[0]
    idx2 = idx.reshape(1, n)
    mesh = plsc.VectorSubcoreMesh(core_axis_name="core",
                                  subcore_axis_name="subcore")

    @pl.kernel(out_type=jax.ShapeDtypeStruct((n, _D), jnp.float32),
               mesh=mesh)
    def k(e_hbm, i_hbm, o_hbm):
        def body(i_vmem, o_vmem):
            pltpu.sync_copy(e_hbm.at[i_vmem.at[0]], o_vmem)

        pltpu.emit_pipeline(
            body,
            grid=(n // _GW,),
            in_specs=[pl.BlockSpec((1, _GW), lambda i: (0, i))],
            out_specs=[pl.BlockSpec((_GW, _D), lambda i: (i, 0))],
            core_axis_name=("core", "subcore"),
            dimension_semantics=(pltpu.PARALLEL,),
        )(i_hbm, o_hbm)

    return k(embed, idx2)


def _transpose_loss_body(q_ref, lp_ref, o_ref, loss_ref):
    b = pl.program_id(0)

    @pl.when(b == 0)
    def _():
        scale = _COMMIT / float(_N_TOK * _D)
        loss_ref[...] = jnp.sum(lp_ref[...], axis=1, keepdims=True) * scale

    o_ref[...] = jnp.transpose(q_ref[...], (0, 2, 1))


def _transpose_loss(qflat, loss_parts, nb):
    q3 = qflat.reshape(nb, _HW, _D)
    return pl.pallas_call(
        _transpose_loss_body,
        grid=(nb,),
        in_specs=[pl.BlockSpec((1, _HW, _D), lambda b: (b, 0, 0)),
                  pl.BlockSpec((1, _NT), lambda b: (0, 0))],
        out_specs=[pl.BlockSpec((1, _D, _HW), lambda b: (b, 0, 0)),
                   pl.BlockSpec((1, 1), lambda b: (0, 0))],
        out_shape=[jax.ShapeDtypeStruct((nb, _D, _HW), jnp.float32),
                   jax.ShapeDtypeStruct((1, 1), jnp.float32)],
    )(q3, loss_parts)


def kernel(z_e, embed_weight):
    B, D, H, W = z_e.shape
    flat = jnp.transpose(z_e, (0, 2, 3, 1)).reshape(-1, D)
    b2d = _bnorm(embed_weight)
    idx, lp = _compute_indices(flat, embed_weight, b2d)
    q = _sc_gather(embed_weight, idx)
    qt, loss2d = _transpose_loss(q, lp, B)
    quantized_st = qt.reshape(B, D, H, W)
    indices = idx.reshape(B, H * W)
    return quantized_st, indices, loss2d.reshape(())


# speedup vs baseline: 1.0112x; 1.0112x over previous
import jax
import jax.numpy as jnp
from jax.experimental import pallas as pl
from jax.experimental.pallas import tpu as pltpu
from jax.experimental.pallas import tpu_sc as plsc

_K = 8192
_D = 256
_TT = 768
_N_TOK = 4608
_NT = _N_TOK // _TT
_HW = 576
_B = 8
_COMMIT = 0.25


def _bnorm_body(e_ref, b_ref):
    e = e_ref[...]
    b_ref[...] = jnp.sum(e * e, axis=1)[None, :]


def _bnorm(embed):
    return pl.pallas_call(
        _bnorm_body,
        grid=(4,),
        in_specs=[pl.BlockSpec((_K // 4, _D), lambda i: (i, 0))],
        out_specs=pl.BlockSpec((1, _K // 4), lambda i: (0, i)),
        out_shape=jax.ShapeDtypeStruct((1, _K), jnp.float32),
    )(embed)


def _dist_body(x_ref, e_ref, b_ref, idx_ref, loss_ref, m_ref):
    x = x_ref[...]
    a = jnp.sum(x * x, axis=1, keepdims=True)
    x2 = x + x
    kt_sz = 512
    for kt in range(_K // kt_sz):
        sk = slice(kt * kt_sz, (kt + 1) * kt_sz)
        m_ref[:, sk] = jax.lax.dot_general(
            x2, e_ref[sk, :], (((1,), (1,)), ((), ())),
            preferred_element_type=jnp.float32)

    nslice = _K // 128
    colmin = (a + b_ref[:, 0:128]) - m_ref[:, 0:128]
    colidx = jnp.zeros((_TT, 128), jnp.int32)
    for j in range(1, nslice):
        sl = slice(j * 128, (j + 1) * 128)
        dj = (a + b_ref[:, sl]) - m_ref[:, sl]
        lt = dj < colmin
        colmin = jnp.minimum(colmin, dj)
        colidx = jnp.where(lt, j, colidx)
    lane = jax.lax.broadcasted_iota(jnp.int32, (_TT, 128), 1)
    g = colidx * 128 + lane
    dmin = jnp.min(colmin, axis=1, keepdims=True)
    idx = jnp.min(jnp.where(colmin == dmin, g, _K), axis=1)
    idx_ref[0, 0, :] = idx
    loss_ref[...] = jnp.sum(dmin, axis=0, keepdims=True)[None]


def _compute_indices(flat, embed, b2d):
    nt = flat.shape[0] // _TT
    idx3, loss = pl.pallas_call(
        _dist_body,
        grid=(nt,),
        in_specs=[
            pl.BlockSpec((_TT, _D), lambda i: (i, 0)),
            pl.BlockSpec((_K, _D), lambda i: (0, 0)),
            pl.BlockSpec((1, _K), lambda i: (0, 0)),
        ],
        out_specs=[
            pl.BlockSpec((1, 1, _TT), lambda i: (i, 0, 0)),
            pl.BlockSpec((1, 1, 1), lambda i: (i, 0, 0)),
        ],
        out_shape=[
            jax.ShapeDtypeStruct((nt, 1, _TT), jnp.int32),
            jax.ShapeDtypeStruct((nt, 1, 1), jnp.float32),
        ],
        scratch_shapes=[pltpu.VMEM((_TT, _K), jnp.float32)],
    )(flat, embed, b2d)
    return idx3.reshape(nt * _TT), loss.reshape(1, nt)


_GW = 128


def _sc_gather(embed, idx):
    n = idx.shape[0]
    idx2 = idx.reshape(1, n)
    mesh = plsc.VectorSubcoreMesh(core_axis_name="core",
                                  subcore_axis_name="subcore")

    @pl.kernel(out_type=jax.ShapeDtypeStruct((n, _D), jnp.float32),
               mesh=mesh)
    def k(e_hbm, i_hbm, o_hbm):
        def body(i_vmem, o_vmem):
            pltpu.sync_copy(e_hbm.at[i_vmem.at[0]], o_vmem)

        pltpu.emit_pipeline(
            body,
            grid=(n // _GW,),
            in_specs=[pl.BlockSpec((1, _GW), lambda i: (0, i))],
            out_specs=[pl.BlockSpec((_GW, _D), lambda i: (i, 0))],
            core_axis_name=("core", "subcore"),
            dimension_semantics=(pltpu.PARALLEL,),
        )(i_hbm, o_hbm)

    return k(embed, idx2)


def _transpose_loss_body(q_ref, lp_ref, o_ref, loss_ref):
    b = pl.program_id(0)

    @pl.when(b == 0)
    def _():
        scale = _COMMIT / float(_N_TOK * _D)
        loss_ref[...] = jnp.sum(lp_ref[...], axis=1, keepdims=True) * scale

    o_ref[...] = jnp.transpose(q_ref[...], (0, 2, 1))


def _transpose_loss(qflat, loss_parts, nb):
    q3 = qflat.reshape(nb, _HW, _D)
    return pl.pallas_call(
        _transpose_loss_body,
        grid=(nb,),
        in_specs=[pl.BlockSpec((1, _HW, _D), lambda b: (b, 0, 0)),
                  pl.BlockSpec((1, _NT), lambda b: (0, 0))],
        out_specs=[pl.BlockSpec((1, _D, _HW), lambda b: (b, 0, 0)),
                   pl.BlockSpec((1, 1), lambda b: (0, 0))],
        out_shape=[jax.ShapeDtypeStruct((nb, _D, _HW), jnp.float32),
                   jax.ShapeDtypeStruct((1, 1), jnp.float32)],
    )(q3, loss_parts)


def kernel(z_e, embed_weight):
    B, D, H, W = z_e.shape
    flat = jnp.transpose(z_e, (0, 2, 3, 1)).reshape(-1, D)
    b2d = _bnorm(embed_weight)
    idx, lp = _compute_indices(flat, embed_weight, b2d)
    q = _sc_gather(embed_weight, idx)
    qt, loss2d = _transpose_loss(q, lp, B)
    quantized_st = qt.reshape(B, D, H, W)
    indices = idx.reshape(B, H * W)
    return quantized_st, indices, loss2d.reshape(())
